# baseline (speedup 1.0000x reference)
import jax
import jax.numpy as jnp
from jax.experimental import pallas as pl

_NUM_EXPERTS = 8
_TOP_K = 2
_HIDDEN = 768
_BLOCK_T = 2048


def _router_body(x_ref, w_ref, b_ref, logits_ref, aff_ref, idx_ref):
    x = x_ref[...]
    w = w_ref[...]
    logits = jnp.dot(x, w, preferred_element_type=jnp.float32) + b_ref[...]
    logits_ref[...] = logits

    m = jnp.max(logits, axis=1, keepdims=True)
    e = jnp.exp(logits - m)
    aff_ref[...] = e / jnp.sum(e, axis=1, keepdims=True)

    iota = jax.lax.broadcasted_iota(jnp.int32, logits.shape, 1)
    sentinel = jnp.int32(_NUM_EXPERTS)
    idx1 = jnp.min(jnp.where(logits == m, iota, sentinel), axis=1, keepdims=True)
    masked = jnp.where(iota == idx1, -jnp.inf, logits)
    m2 = jnp.max(masked, axis=1, keepdims=True)
    idx2 = jnp.min(jnp.where(masked == m2, iota, sentinel), axis=1, keepdims=True)
    idx_ref[...] = jnp.concatenate([idx1, idx2], axis=1)


def kernel(hidden_states, W, b):
    S, B, H = hidden_states.shape
    T = S * B
    x = hidden_states.reshape(T, H)
    wt = W.T
    b2 = b.reshape(1, _NUM_EXPERTS)

    grid = (T // _BLOCK_T,)
    logits, aff, idx = pl.pallas_call(
        _router_body,
        grid=grid,
        in_specs=[
            pl.BlockSpec((_BLOCK_T, H), lambda i: (i, 0)),
            pl.BlockSpec((H, _NUM_EXPERTS), lambda i: (0, 0)),
            pl.BlockSpec((1, _NUM_EXPERTS), lambda i: (0, 0)),
        ],
        out_specs=[
            pl.BlockSpec((_BLOCK_T, _NUM_EXPERTS), lambda i: (i, 0)),
            pl.BlockSpec((_BLOCK_T, _NUM_EXPERTS), lambda i: (i, 0)),
            pl.BlockSpec((_BLOCK_T, _TOP_K), lambda i: (i, 0)),
        ],
        out_shape=[
            jax.ShapeDtypeStruct((T, _NUM_EXPERTS), jnp.float32),
            jax.ShapeDtypeStruct((T, _NUM_EXPERTS), jnp.float32),
            jax.ShapeDtypeStruct((T, _TOP_K), jnp.int32),
        ],
    )(x, wt, b2)
    return (logits, aff, idx)


# speedup vs baseline: 1.3524x; 1.3524x over previous
import jax
import jax.numpy as jnp
from jax.experimental import pallas as pl

_NUM_EXPERTS = 8
_TOP_K = 2
_BLOCK_T = 2048


def _router_body(x_ref, w_ref, b_ref, logits_ref, aff_ref, idx_ref):
    x = x_ref[...]
    w = w_ref[...]
    logits = jnp.dot(x, w, preferred_element_type=jnp.float32) + b_ref[...]
    lt = logits.T
    logits_ref[...] = lt

    m = jnp.max(lt, axis=0, keepdims=True)
    e = jnp.exp(lt - m)
    aff_ref[...] = e / jnp.sum(e, axis=0, keepdims=True)

    iota = jax.lax.broadcasted_iota(jnp.int32, lt.shape, 0)
    sentinel = jnp.int32(_NUM_EXPERTS)
    idx1 = jnp.min(jnp.where(lt == m, iota, sentinel), axis=0, keepdims=True)
    masked = jnp.where(iota == idx1, -jnp.inf, lt)
    m2 = jnp.max(masked, axis=0, keepdims=True)
    idx2 = jnp.min(jnp.where(masked == m2, iota, sentinel), axis=0, keepdims=True)
    idx_ref[...] = jnp.concatenate([idx1, idx2], axis=0)


def kernel(hidden_states, W, b):
    S, B, H = hidden_states.shape
    T = S * B
    x = hidden_states.reshape(T, H)
    wt = W.T
    b2 = b.reshape(1, _NUM_EXPERTS)

    grid = (T // _BLOCK_T,)
    logits_t, aff_t, idx_t = pl.pallas_call(
        _router_body,
        grid=grid,
        in_specs=[
            pl.BlockSpec((_BLOCK_T, H), lambda i: (i, 0)),
            pl.BlockSpec((H, _NUM_EXPERTS), lambda i: (0, 0)),
            pl.BlockSpec((1, _NUM_EXPERTS), lambda i: (0, 0)),
        ],
        out_specs=[
            pl.BlockSpec((_NUM_EXPERTS, _BLOCK_T), lambda i: (0, i)),
            pl.BlockSpec((_NUM_EXPERTS, _BLOCK_T), lambda i: (0, i)),
            pl.BlockSpec((_TOP_K, _BLOCK_T), lambda i: (0, i)),
        ],
        out_shape=[
            jax.ShapeDtypeStruct((_NUM_EXPERTS, T), jnp.float32),
            jax.ShapeDtypeStruct((_NUM_EXPERTS, T), jnp.float32),
            jax.ShapeDtypeStruct((_TOP_K, T), jnp.int32),
        ],
    )(x, wt, b2)
    return (logits_t.T, aff_t.T, idx_t.T)


# speedup vs baseline: 5.1777x; 3.8286x over previous
import jax
import jax.numpy as jnp
from jax.experimental import pallas as pl

_NUM_EXPERTS = 8
_TOP_K = 2
_BLOCK_S = 512


def _router_body(x_ref, w_ref, b_ref, logits_ref, aff_ref, idx_ref):
    bs, bdim, h = x_ref.shape
    x = x_ref[...].reshape(bs * bdim, h)
    w = w_ref[...]
    logits = jnp.dot(x, w, preferred_element_type=jnp.float32) + b_ref[...]
    lt = logits.T
    logits_ref[...] = lt

    m = jnp.max(lt, axis=0, keepdims=True)
    e = jnp.exp(lt - m)
    aff_ref[...] = e / jnp.sum(e, axis=0, keepdims=True)

    iota = jax.lax.broadcasted_iota(jnp.int32, lt.shape, 0)
    sentinel = jnp.int32(_NUM_EXPERTS)
    idx1 = jnp.min(jnp.where(lt == m, iota, sentinel), axis=0, keepdims=True)
    masked = jnp.where(iota == idx1, -jnp.inf, lt)
    m2 = jnp.max(masked, axis=0, keepdims=True)
    idx2 = jnp.min(jnp.where(masked == m2, iota, sentinel), axis=0, keepdims=True)
    idx_ref[...] = jnp.concatenate([idx1, idx2], axis=0)


def kernel(hidden_states, W, b):
    S, B, H = hidden_states.shape
    T = S * B
    block_t = _BLOCK_S * B
    wt = W.T
    b2 = b.reshape(1, _NUM_EXPERTS)

    grid = (S // _BLOCK_S,)
    logits_t, aff_t, idx_t = pl.pallas_call(
        _router_body,
        grid=grid,
        in_specs=[
            pl.BlockSpec((_BLOCK_S, B, H), lambda i: (i, 0, 0)),
            pl.BlockSpec((H, _NUM_EXPERTS), lambda i: (0, 0)),
            pl.BlockSpec((1, _NUM_EXPERTS), lambda i: (0, 0)),
        ],
        out_specs=[
            pl.BlockSpec((_NUM_EXPERTS, block_t), lambda i: (0, i)),
            pl.BlockSpec((_NUM_EXPERTS, block_t), lambda i: (0, i)),
            pl.BlockSpec((_TOP_K, block_t), lambda i: (0, i)),
        ],
        out_shape=[
            jax.ShapeDtypeStruct((_NUM_EXPERTS, T), jnp.float32),
            jax.ShapeDtypeStruct((_NUM_EXPERTS, T), jnp.float32),
            jax.ShapeDtypeStruct((_TOP_K, T), jnp.int32),
        ],
    )(hidden_states, wt, b2)
    return (logits_t.T, aff_t.T, idx_t.T)


# speedup vs baseline: 5.2904x; 1.0218x over previous
import jax
import jax.numpy as jnp
from jax.experimental import pallas as pl

_NUM_EXPERTS = 8
_TOP_K = 2
_BLOCK_S = 1024


def _router_body(x_ref, w_ref, b_ref, logits_ref, aff_ref, idx_ref):
    bs, bdim, h = x_ref.shape
    x = x_ref[...].reshape(bs * bdim, h)
    w = w_ref[...]
    logits = jnp.dot(x, w, preferred_element_type=jnp.float32) + b_ref[...]
    lt = logits.T
    logits_ref[...] = lt

    m = jnp.max(lt, axis=0, keepdims=True)
    e = jnp.exp(lt - m)
    aff_ref[...] = e / jnp.sum(e, axis=0, keepdims=True)

    iota = jax.lax.broadcasted_iota(jnp.int32, lt.shape, 0)
    sentinel = jnp.int32(_NUM_EXPERTS)
    idx1 = jnp.min(jnp.where(lt == m, iota, sentinel), axis=0, keepdims=True)
    masked = jnp.where(iota == idx1, -jnp.inf, lt)
    m2 = jnp.max(masked, axis=0, keepdims=True)
    idx2 = jnp.min(jnp.where(masked == m2, iota, sentinel), axis=0, keepdims=True)
    idx_ref[...] = jnp.concatenate([idx1, idx2], axis=0)


def kernel(hidden_states, W, b):
    S, B, H = hidden_states.shape
    T = S * B
    block_t = _BLOCK_S * B
    wt = W.T
    b2 = b.reshape(1, _NUM_EXPERTS)

    grid = (S // _BLOCK_S,)
    logits_t, aff_t, idx_t = pl.pallas_call(
        _router_body,
        grid=grid,
        in_specs=[
            pl.BlockSpec((_BLOCK_S, B, H), lambda i: (i, 0, 0)),
            pl.BlockSpec((H, _NUM_EXPERTS), lambda i: (0, 0)),
            pl.BlockSpec((1, _NUM_EXPERTS), lambda i: (0, 0)),
        ],
        out_specs=[
            pl.BlockSpec((_NUM_EXPERTS, block_t), lambda i: (0, i)),
            pl.BlockSpec((_NUM_EXPERTS, block_t), lambda i: (0, i)),
            pl.BlockSpec((_TOP_K, block_t), lambda i: (0, i)),
        ],
        out_shape=[
            jax.ShapeDtypeStruct((_NUM_EXPERTS, T), jnp.float32),
            jax.ShapeDtypeStruct((_NUM_EXPERTS, T), jnp.float32),
            jax.ShapeDtypeStruct((_TOP_K, T), jnp.int32),
        ],
    )(hidden_states, wt, b2)
    return (logits_t.T, aff_t.T, idx_t.T)
